# baseline (speedup 1.0000x reference)
import functools

import jax
import jax.numpy as jnp
from jax import lax
from jax.experimental import pallas as pl
from jax.experimental.pallas import tpu as pltpu
from jax.experimental.pallas import tpu_sc as plsc

_L = 16


@functools.lru_cache(maxsize=None)
def _make_sc_kernel(B, C, H, W):
    E = C // 2
    NC, NS = 2, 16
    NW = NC * NS
    CPW = C // NW
    assert C % NW == 0 and E % CPW == 0 and W % _L == 0
    PLANE = H * W
    BLK = CPW * PLANE
    BSTRIDE = C * PLANE
    TAB = 2 * H * E

    mesh = plsc.VectorSubcoreMesh(core_axis_name="c", subcore_axis_name="s")

    @functools.partial(
        pl.kernel,
        mesh=mesh,
        compiler_params=pltpu.CompilerParams(needs_layout_passes=False),
        out_type=jax.ShapeDtypeStruct((B * BSTRIDE,), jnp.float32),
        scratch_types=[
            pltpu.VMEM((TAB,), jnp.float32),
            pltpu.VMEM((BLK,), jnp.float32),
            pltpu.SemaphoreType.DMA,
        ],
    )
    def k(tab_hbm, out_hbm, tab_v, buf_v, sem):
        wid = lax.axis_index("s") * NC + lax.axis_index("c")
        pltpu.sync_copy(tab_hbm, tab_v)
        is_col = wid * CPW < E
        pred_v = jnp.broadcast_to(is_col, (_L,))
        cbase = jnp.where(is_col, wid * CPW, wid * CPW - E).astype(jnp.int32)
        for cc in range(CPW):
            cvec = jnp.broadcast_to(cbase + cc, (_L,))
            for i in range(H):
                row_idx = jnp.full((_L,), (H + i) * E, jnp.int32) + cvec
                for j0 in range(W // _L):
                    col_idx = (lax.iota(jnp.int32, _L) + j0 * _L) * E + cvec
                    idx = jnp.where(pred_v, col_idx, row_idx)
                    vals = plsc.load_gather(tab_v, [idx])
                    buf_v[pl.ds(cc * PLANE + i * W + j0 * _L, _L)] = vals
        copies = [
            pltpu.async_copy(
                buf_v, out_hbm.at[pl.ds(b * BSTRIDE + wid * BLK, BLK)], sem)
            for b in range(B)
        ]
        for cp in copies:
            cp.wait()

    return k


def kernel(x, row_embed, col_embed):
    B, C, H, W = x.shape
    E = C // 2
    tab = jnp.concatenate(
        [col_embed[:H, :E].reshape(-1), row_embed[:H, :E].reshape(-1)])
    flat = _make_sc_kernel(B, C, H, W)(tab)
    return flat.reshape(B, C, H, W)


# speedup vs baseline: 2.5796x; 2.5796x over previous
import functools

import jax
import jax.numpy as jnp
from jax import lax
from jax.experimental import pallas as pl
from jax.experimental.pallas import tpu as pltpu
from jax.experimental.pallas import tpu_sc as plsc

_L = 16


@functools.lru_cache(maxsize=None)
def _make_sc_kernel(B, C, H, W):
    E = C // 2
    NC, NS = 2, 16
    NW = NC * NS
    assert H == NW and E % _L == 0 and C == 2 * E

    mesh = plsc.VectorSubcoreMesh(core_axis_name="c", subcore_axis_name="s")

    @functools.partial(
        pl.kernel,
        mesh=mesh,
        compiler_params=pltpu.CompilerParams(needs_layout_passes=False),
        out_type=jax.ShapeDtypeStruct((B, H, W, C), jnp.float32),
        scratch_types=[
            pltpu.VMEM((2 * H * E,), jnp.float32),
            pltpu.VMEM((W, C), jnp.float32),
            pltpu.SemaphoreType.DMA,
        ],
    )
    def k(tab_hbm, out_hbm, tab_v, slab_v, sem):
        wid = lax.axis_index("s") * NC + lax.axis_index("c")
        pltpu.sync_copy(tab_hbm, tab_v)
        for j in range(W):
            for c0 in range(0, E, _L):
                slab_v[j, pl.ds(c0, _L)] = tab_v[pl.ds(j * E + c0, _L)]
        base = jnp.broadcast_to(H * E + wid * E, (_L,)).astype(jnp.int32)
        for c0 in range(0, E, _L):
            rv = plsc.load_gather(tab_v, [base + c0 + lax.iota(jnp.int32, _L)])
            for j in range(W):
                slab_v[j, pl.ds(E + c0, _L)] = rv
        copies = [pltpu.async_copy(slab_v, out_hbm.at[b, wid], sem)
                  for b in range(B)]
        for cp in copies:
            cp.wait()

    return k


def kernel(x, row_embed, col_embed):
    B, C, H, W = x.shape
    E = C // 2
    tab = jnp.concatenate(
        [col_embed[:W, :E].reshape(-1), row_embed[:H, :E].reshape(-1)])
    out = _make_sc_kernel(B, C, H, W)(tab)
    return out.transpose(0, 3, 1, 2)


# speedup vs baseline: 2.6014x; 1.0084x over previous
import functools

import jax
import jax.numpy as jnp
from jax import lax
from jax.experimental import pallas as pl
from jax.experimental.pallas import tpu as pltpu
from jax.experimental.pallas import tpu_sc as plsc

_L = 16


@functools.lru_cache(maxsize=None)
def _make_sc_kernel(B, C, H, W):
    E = C // 2
    NC, NS = 2, 16
    NW = NC * NS
    assert H == NW and W == NW and E % _L == 0 and C == 2 * E

    mesh = plsc.VectorSubcoreMesh(core_axis_name="c", subcore_axis_name="s")

    @functools.partial(
        pl.kernel,
        mesh=mesh,
        compiler_params=pltpu.CompilerParams(needs_layout_passes=False),
        out_type=jax.ShapeDtypeStruct((B, H, W, C), jnp.float32),
        scratch_types=[
            pltpu.VMEM((H, E), jnp.float32),
            pltpu.VMEM((W, C), jnp.float32),
            pltpu.SemaphoreType.DMA,
        ],
    )
    def k(row_hbm, col_hbm, out_hbm, rowt_v, slab_v, sem):
        wid = lax.axis_index("s") * NC + lax.axis_index("c")
        cp_col = pltpu.async_copy(
            col_hbm.at[pl.ds(0, W)], slab_v.at[:, pl.ds(0, E)], sem)
        cp_row = pltpu.async_copy(row_hbm.at[pl.ds(0, H)], rowt_v, sem)
        cp_col.wait()
        cp_row.wait()
        wv = jnp.broadcast_to(wid, (_L,)).astype(jnp.int32)
        for c0 in range(0, E, _L):
            rv = plsc.load_gather(rowt_v, [wv, c0 + lax.iota(jnp.int32, _L)])
            for j in range(W):
                slab_v[j, pl.ds(E + c0, _L)] = rv
        copies = [pltpu.async_copy(slab_v, out_hbm.at[b, wid], sem)
                  for b in range(B)]
        for cp in copies:
            cp.wait()

    return k


def kernel(x, row_embed, col_embed):
    B, C, H, W = x.shape
    out = _make_sc_kernel(B, C, H, W)(row_embed, col_embed)
    return out.transpose(0, 3, 1, 2)


# speedup vs baseline: 2.6026x; 1.0005x over previous
import functools

import jax
import jax.numpy as jnp
from jax import lax
from jax.experimental import pallas as pl
from jax.experimental.pallas import tpu as pltpu
from jax.experimental.pallas import tpu_sc as plsc

_L = 16


@functools.lru_cache(maxsize=None)
def _make_sc_kernel(B, C, H, W):
    E = C // 2
    NC, NS = 2, 16
    NW = NC * NS
    assert H == NW and W == NW and E % _L == 0 and C == 2 * E

    mesh = plsc.VectorSubcoreMesh(core_axis_name="c", subcore_axis_name="s")

    @functools.partial(
        pl.kernel,
        mesh=mesh,
        compiler_params=pltpu.CompilerParams(needs_layout_passes=False),
        out_type=jax.ShapeDtypeStruct((B, H, W, C), jnp.float32),
        scratch_types=[
            pltpu.VMEM((H, E), jnp.float32),
            pltpu.VMEM((W, C), jnp.float32),
            pltpu.SemaphoreType.DMA,
            pltpu.SemaphoreType.DMA,
        ],
    )
    def k(row_hbm, col_hbm, out_hbm, rowt_v, slab_v, sem, sem2):
        wid = lax.axis_index("c") * NS + lax.axis_index("s")
        cp_col = pltpu.async_copy(
            col_hbm.at[pl.ds(0, W)], slab_v.at[:, pl.ds(0, E)], sem)
        cp_row = pltpu.async_copy(row_hbm.at[pl.ds(0, H)], rowt_v, sem2)
        cp_row.wait()
        wv = jnp.broadcast_to(wid, (_L,)).astype(jnp.int32)
        for c0 in range(0, E, _L):
            rv = plsc.load_gather(rowt_v, [wv, c0 + lax.iota(jnp.int32, _L)])
            for j in range(W):
                slab_v[j, pl.ds(E + c0, _L)] = rv
        cp_col.wait()
        copies = [pltpu.async_copy(slab_v, out_hbm.at[b, wid], sem)
                  for b in range(B)]
        for cp in copies:
            cp.wait()

    return k


def kernel(x, row_embed, col_embed):
    B, C, H, W = x.shape
    out = _make_sc_kernel(B, C, H, W)(row_embed, col_embed)
    return out.transpose(0, 3, 1, 2)
